# baseline (speedup 1.0000x reference)
import jax
import jax.numpy as jnp
from jax import lax
from jax.experimental import pallas as pl
from jax.experimental.pallas import tpu as pltpu
from jax.experimental.pallas import tpu_sc as plsc

N = 10000
D = 128
HH = 256
NT = 2
E = 320000
HALF = D // 2
TILES = 16
PER_TILE = E // TILES
K = 80
CH = 2
NCH = 130
NB = CH * NCH
EPT_PAD = NB * K
SEG = 20480
ROWS_PER_TILE = SEG // TILES
BN = 1000
GRID = N // BN


def _edge_mlp_body(nf_ref, w1_ref, b1_ref, w2_ref, b2_ref, out_ref):
    x = nf_ref[...]
    for t in range(NT):
        h = jnp.maximum(
            jnp.dot(x, w1_ref[t], preferred_element_type=jnp.float32) + b1_ref[t],
            0.0)
        m = jnp.maximum(
            jnp.dot(h, w2_ref[t], preferred_element_type=jnp.float32) + b2_ref[t],
            0.0)
        out_ref[t] = m


def _edge_mlp(nf, w1s, b1s, w2s, b2s, interpret=False):
    return pl.pallas_call(
        _edge_mlp_body,
        grid=(GRID,),
        in_specs=[
            pl.BlockSpec((BN, D), lambda j: (j, 0)),
            pl.BlockSpec((NT, D, HH), lambda j: (0, 0, 0)),
            pl.BlockSpec((NT, 1, HH), lambda j: (0, 0, 0)),
            pl.BlockSpec((NT, HH, D), lambda j: (0, 0, 0)),
            pl.BlockSpec((NT, 1, D), lambda j: (0, 0, 0)),
        ],
        out_specs=pl.BlockSpec((NT, BN, D), lambda j: (0, j, 0)),
        out_shape=jax.ShapeDtypeStruct((NT, N, D), jnp.float32),
        interpret=interpret,
    )(nf, w1s, b1s, w2s, b2s)


def _node_mlp_body(nf_ref, a00_ref, a01_ref, a10_ref, a11_ref,
                   w1_ref, b1_ref, w2_ref, b2_ref, out_ref):
    x = jnp.maximum(nf_ref[...], 0.0)
    h = jnp.dot(x, w1_ref[0:D], preferred_element_type=jnp.float32)
    h += jnp.dot(a00_ref[0], w1_ref[D:D + HALF],
                 preferred_element_type=jnp.float32)
    h += jnp.dot(a01_ref[0], w1_ref[D + HALF:2 * D],
                 preferred_element_type=jnp.float32)
    h += jnp.dot(a10_ref[0], w1_ref[2 * D:2 * D + HALF],
                 preferred_element_type=jnp.float32)
    h += jnp.dot(a11_ref[0], w1_ref[2 * D + HALF:3 * D],
                 preferred_element_type=jnp.float32)
    h = jnp.maximum(h + b1_ref[...], 0.0)
    out_ref[...] = (jnp.dot(h, w2_ref[...], preferred_element_type=jnp.float32)
                    + b2_ref[...])


def _node_mlp(nf, agg, w1, b1, w2, b2, interpret=False):
    return pl.pallas_call(
        _node_mlp_body,
        grid=(GRID,),
        in_specs=[
            pl.BlockSpec((BN, D), lambda j: (j, 0)),
            pl.BlockSpec((1, BN, HALF), lambda j: (0, j, 0)),
            pl.BlockSpec((1, BN, HALF), lambda j: (1, j, 0)),
            pl.BlockSpec((1, BN, HALF), lambda j: (0, GRID + j, 0)),
            pl.BlockSpec((1, BN, HALF), lambda j: (1, GRID + j, 0)),
            pl.BlockSpec((3 * D, HH), lambda j: (0, 0)),
            pl.BlockSpec((1, HH), lambda j: (0, 0)),
            pl.BlockSpec((HH, D), lambda j: (0, 0)),
            pl.BlockSpec((1, D), lambda j: (0, 0)),
        ],
        out_specs=pl.BlockSpec((BN, D), lambda j: (j, 0)),
        out_shape=jax.ShapeDtypeStruct((N, D), jnp.float32),
        interpret=interpret,
    )(nf, agg, agg, agg, agg, w1, b1, w2, b2)


def _sc_agg_body(m4_hbm, gidx_hbm, sidx_hbm, zeros_hbm, out_hbm,
                 agg_sh, gvc, svc, rows, semi, semg):
    c = lax.axis_index("c")
    s = lax.axis_index("s")
    pltpu.sync_copy(zeros_hbm, agg_sh.at[pl.ds(s * ROWS_PER_TILE, ROWS_PER_TILE)])
    plsc.subcore_barrier()

    def fetch_chunk(ch, p):
        pltpu.async_copy(gidx_hbm.at[c, s, pl.ds(ch * CH, CH)], gvc[p], semi[p])
        pltpu.async_copy(sidx_hbm.at[s, pl.ds(ch * CH, CH)], svc[p], semi[p])

    def wait_chunk(ch, p):
        pltpu.make_async_copy(gidx_hbm.at[c, s, pl.ds(ch * CH, CH)],
                              gvc[p], semi[p]).wait()
        pltpu.make_async_copy(sidx_hbm.at[s, pl.ds(ch * CH, CH)],
                              svc[p], semi[p]).wait()

    def gather(p, u, j):
        pltpu.async_copy(m4_hbm.at[gvc[p].at[u]], rows[j], semg[j])

    def wait_gather(p, u, j):
        pltpu.make_async_copy(m4_hbm.at[gvc[p].at[u]], rows[j], semg[j]).wait()

    fetch_chunk(0, 0)
    fetch_chunk(1, 1)
    wait_chunk(0, 0)
    gather(0, 0, 0)

    def chunk_step(ch, p):
        q = 1 - p
        for u in range(CH):
            j = u % 2
            j1 = (u + 1) % 2
            if u < CH - 1:
                gather(p, u + 1, j1)
            else:
                @pl.when(ch + 1 < NCH)
                def _():
                    wait_chunk(ch + 1, q)
                    gather(q, 0, j1)
            wait_gather(p, u, j)
            pltpu.sync_copy(rows[j], agg_sh.at[svc[p].at[u]], add=True)

        @pl.when(ch + 2 < NCH)
        def _():
            fetch_chunk(ch + 2, p)

    def body(i, carry):
        ch0 = 2 * i
        chunk_step(ch0, 0)
        chunk_step(ch0 + 1, 1)
        return carry

    lax.fori_loop(0, NCH // 2, body, 0)
    plsc.subcore_barrier()
    pltpu.sync_copy(agg_sh.at[pl.ds(s * ROWS_PER_TILE, ROWS_PER_TILE)],
                    out_hbm.at[c, pl.ds(s * ROWS_PER_TILE, ROWS_PER_TILE)])


def _sc_agg(m4, gidx, sidx, zeros):
    mesh = plsc.VectorSubcoreMesh(core_axis_name="c", subcore_axis_name="s")
    kern = pl.kernel(
        _sc_agg_body,
        out_type=jax.ShapeDtypeStruct((NT, SEG, HALF), jnp.float32),
        mesh=mesh,
        scratch_types=[
            pltpu.VMEM_SHARED((SEG, HALF), jnp.float32),
            [pltpu.VMEM((CH, K), jnp.int32) for _ in range(2)],
            [pltpu.VMEM((CH, K), jnp.int32) for _ in range(2)],
            [pltpu.VMEM((K, HALF), jnp.float32) for _ in range(2)],
            [pltpu.SemaphoreType.DMA for _ in range(2)],
            [pltpu.SemaphoreType.DMA for _ in range(2)],
        ],
        compiler_params=pltpu.CompilerParams(use_tc_tiling_on_sc=False),
    )
    return kern(m4, gidx, sidx, zeros)


def kernel(x, edge_index, edge_type, params):
    src = edge_index[0].astype(jnp.int32)
    dst = edge_index[1].astype(jnp.int32)
    et = edge_type.astype(jnp.int32)
    base = et * N + src
    pad = EPT_PAD - PER_TILE
    gidx = jnp.pad(
        jnp.stack([base * 2, base * 2 + 1]).reshape(NT, TILES, PER_TILE),
        ((0, 0), (0, 0), (0, pad))).reshape(NT, TILES, NB, K)
    sidx = jnp.pad((et * N + dst).reshape(TILES, PER_TILE),
                   ((0, 0), (0, pad)),
                   constant_values=NT * N).reshape(TILES, NB, K)
    zeros = jnp.zeros((ROWS_PER_TILE, HALF), jnp.float32)

    nf = x
    for lp in params:
        w1s = jnp.stack([lp["edge0"]["W1"], lp["edge1"]["W1"]])
        b1s = jnp.stack([lp["edge0"]["b1"], lp["edge1"]["b1"]])[:, None, :]
        w2s = jnp.stack([lp["edge0"]["W2"], lp["edge1"]["W2"]])
        b2s = jnp.stack([lp["edge0"]["b2"], lp["edge1"]["b2"]])[:, None, :]
        m = _edge_mlp(nf, w1s, b1s, w2s, b2s)
        m4 = m.reshape(2 * NT * N, HALF)
        agg = _sc_agg(m4, gidx, sidx, zeros)
        npar = lp["node"]
        nf = _node_mlp(nf, agg, npar["W1"], npar["b1"][None, :],
                       npar["W2"], npar["b2"][None, :])
    return nf


# speedup vs baseline: 3.1579x; 3.1579x over previous
import jax
import jax.numpy as jnp
from jax import lax
from jax.experimental import pallas as pl
from jax.experimental.pallas import tpu as pltpu
from jax.experimental.pallas import tpu_sc as plsc

N = 10000
D = 128
HH = 256
NT = 2
E = 320000
HALF = D // 2
TILES = 16
PER_TILE = E // TILES
K = 80
NB = 250
EPT_PAD = NB * K
SEG = 20096
ROWS_PER_TILE = SEG // TILES
BN = 1000
GRID = N // BN


def _edge_mlp_body(nf_ref, w1_ref, b1_ref, w2_ref, b2_ref, out_ref):
    x = nf_ref[...]
    for t in range(NT):
        h = jnp.maximum(
            jnp.dot(x, w1_ref[t], preferred_element_type=jnp.float32) + b1_ref[t],
            0.0)
        m = jnp.maximum(
            jnp.dot(h, w2_ref[t], preferred_element_type=jnp.float32) + b2_ref[t],
            0.0)
        out_ref[t] = m


def _edge_mlp(nf, w1s, b1s, w2s, b2s, interpret=False):
    return pl.pallas_call(
        _edge_mlp_body,
        grid=(GRID,),
        in_specs=[
            pl.BlockSpec((BN, D), lambda j: (j, 0)),
            pl.BlockSpec((NT, D, HH), lambda j: (0, 0, 0)),
            pl.BlockSpec((NT, 1, HH), lambda j: (0, 0, 0)),
            pl.BlockSpec((NT, HH, D), lambda j: (0, 0, 0)),
            pl.BlockSpec((NT, 1, D), lambda j: (0, 0, 0)),
        ],
        out_specs=pl.BlockSpec((NT, BN, D), lambda j: (0, j, 0)),
        out_shape=jax.ShapeDtypeStruct((NT, N, D), jnp.float32),
        interpret=interpret,
    )(nf, w1s, b1s, w2s, b2s)


def _node_mlp_body(nf_ref, a00_ref, a01_ref, a10_ref, a11_ref,
                   w1_ref, b1_ref, w2_ref, b2_ref, out_ref):
    x = jnp.maximum(nf_ref[...], 0.0)
    h = jnp.dot(x, w1_ref[0:D], preferred_element_type=jnp.float32)
    h += jnp.dot(a00_ref[0], w1_ref[D:D + HALF],
                 preferred_element_type=jnp.float32)
    h += jnp.dot(a01_ref[0], w1_ref[D + HALF:2 * D],
                 preferred_element_type=jnp.float32)
    h += jnp.dot(a10_ref[0], w1_ref[2 * D:2 * D + HALF],
                 preferred_element_type=jnp.float32)
    h += jnp.dot(a11_ref[0], w1_ref[2 * D + HALF:3 * D],
                 preferred_element_type=jnp.float32)
    h = jnp.maximum(h + b1_ref[...], 0.0)
    out_ref[...] = (jnp.dot(h, w2_ref[...], preferred_element_type=jnp.float32)
                    + b2_ref[...])


def _node_mlp(nf, agg, w1, b1, w2, b2, interpret=False):
    return pl.pallas_call(
        _node_mlp_body,
        grid=(GRID,),
        in_specs=[
            pl.BlockSpec((BN, D), lambda j: (j, 0)),
            pl.BlockSpec((1, BN, HALF), lambda j: (0, j, 0)),
            pl.BlockSpec((1, BN, HALF), lambda j: (1, j, 0)),
            pl.BlockSpec((1, BN, HALF), lambda j: (0, GRID + j, 0)),
            pl.BlockSpec((1, BN, HALF), lambda j: (1, GRID + j, 0)),
            pl.BlockSpec((3 * D, HH), lambda j: (0, 0)),
            pl.BlockSpec((1, HH), lambda j: (0, 0)),
            pl.BlockSpec((HH, D), lambda j: (0, 0)),
            pl.BlockSpec((1, D), lambda j: (0, 0)),
        ],
        out_specs=pl.BlockSpec((BN, D), lambda j: (j, 0)),
        out_shape=jax.ShapeDtypeStruct((N, D), jnp.float32),
        interpret=interpret,
    )(nf, agg, agg, agg, agg, w1, b1, w2, b2)


def _sc_agg_body(m4_hbm, gidx_hbm, sidx_hbm, zeros_hbm, out_hbm,
                 agg_sh, gvc, svc, rows, semg):
    c = lax.axis_index("c")
    s = lax.axis_index("s")
    pltpu.sync_copy(zeros_hbm, agg_sh.at[pl.ds(s * ROWS_PER_TILE, ROWS_PER_TILE)])
    plsc.subcore_barrier()

    pltpu.sync_copy(gidx_hbm.at[c, s], gvc)
    pltpu.sync_copy(sidx_hbm.at[s], svc)

    def gather(b, j):
        pltpu.async_copy(m4_hbm.at[gvc.at[b]], rows[j], semg[j])

    def wait_gather(b, j):
        pltpu.make_async_copy(m4_hbm.at[gvc.at[b]], rows[j], semg[j]).wait()

    gather(0, 0)

    def step(b, j):
        j1 = (j + 1) % 2

        @pl.when(b + 1 < NB)
        def _():
            gather(b + 1, j1)

        wait_gather(b, j)
        pltpu.sync_copy(rows[j], agg_sh.at[svc.at[b]], add=True)

    def body(i, carry):
        b0 = 2 * i
        step(b0, 0)
        step(b0 + 1, 1)
        return carry

    lax.fori_loop(0, NB // 2, body, 0)
    plsc.subcore_barrier()
    pltpu.sync_copy(agg_sh.at[pl.ds(s * ROWS_PER_TILE, ROWS_PER_TILE)],
                    out_hbm.at[c, pl.ds(s * ROWS_PER_TILE, ROWS_PER_TILE)])


def _sc_agg(m4, gidx, sidx, zeros):
    mesh = plsc.VectorSubcoreMesh(core_axis_name="c", subcore_axis_name="s")
    kern = pl.kernel(
        _sc_agg_body,
        out_type=jax.ShapeDtypeStruct((NT, SEG, HALF), jnp.float32),
        mesh=mesh,
        scratch_types=[
            pltpu.VMEM_SHARED((SEG, HALF), jnp.float32),
            pltpu.VMEM((NB, K), jnp.int32),
            pltpu.VMEM((NB, K), jnp.int32),
            [pltpu.VMEM((K, HALF), jnp.float32) for _ in range(2)],
            [pltpu.SemaphoreType.DMA for _ in range(2)],
        ],
        compiler_params=pltpu.CompilerParams(use_tc_tiling_on_sc=False),
    )
    return kern(m4, gidx, sidx, zeros)


def kernel(x, edge_index, edge_type, params):
    src = edge_index[0].astype(jnp.int32)
    dst = edge_index[1].astype(jnp.int32)
    et = edge_type.astype(jnp.int32)
    base = et * N + src
    pad = EPT_PAD - PER_TILE
    gidx = jnp.pad(
        jnp.stack([base * 2, base * 2 + 1]).reshape(NT, TILES, PER_TILE),
        ((0, 0), (0, 0), (0, pad))).reshape(NT, TILES, NB, K)
    sidx = jnp.pad((et * N + dst).reshape(TILES, PER_TILE),
                   ((0, 0), (0, pad)),
                   constant_values=NT * N).reshape(TILES, NB, K)
    zeros = jnp.zeros((ROWS_PER_TILE, HALF), jnp.float32)

    nf = x
    for lp in params:
        w1s = jnp.stack([lp["edge0"]["W1"], lp["edge1"]["W1"]])
        b1s = jnp.stack([lp["edge0"]["b1"], lp["edge1"]["b1"]])[:, None, :]
        w2s = jnp.stack([lp["edge0"]["W2"], lp["edge1"]["W2"]])
        b2s = jnp.stack([lp["edge0"]["b2"], lp["edge1"]["b2"]])[:, None, :]
        m = _edge_mlp(nf, w1s, b1s, w2s, b2s)
        m4 = m.reshape(2 * NT * N, HALF)
        agg = _sc_agg(m4, gidx, sidx, zeros)
        npar = lp["node"]
        nf = _node_mlp(nf, agg, npar["W1"], npar["b1"][None, :],
                       npar["W2"], npar["b2"][None, :])
    return nf


# speedup vs baseline: 3.2022x; 1.0140x over previous
import jax
import jax.numpy as jnp
from jax import lax
from jax.experimental import pallas as pl
from jax.experimental.pallas import tpu as pltpu
from jax.experimental.pallas import tpu_sc as plsc

N = 10000
D = 128
HH = 256
NT = 2
E = 320000
HALF = D // 2
TILES = 16
PER_TILE = E // TILES
K = 80
NB = 250
EPT_PAD = NB * K
SEG = 20096
ROWS_PER_TILE = SEG // TILES
BN = 1000
GRID = N // BN


def _edge_block(x, ep, t, out_ref):
    h = jnp.maximum(
        jnp.dot(x, ep[4 * t][...], preferred_element_type=jnp.float32)
        + ep[4 * t + 1][...], 0.0)
    m = jnp.maximum(
        jnp.dot(h, ep[4 * t + 2][...], preferred_element_type=jnp.float32)
        + ep[4 * t + 3][...], 0.0)
    out_ref[t] = m


_EDGE_PARAM_SPECS = [
    pl.BlockSpec((D, HH), lambda j: (0, 0)),
    pl.BlockSpec((1, HH), lambda j: (0, 0)),
    pl.BlockSpec((HH, D), lambda j: (0, 0)),
    pl.BlockSpec((1, D), lambda j: (0, 0)),
] * NT


def _edge_args(lp):
    out = []
    for t in range(NT):
        p = lp["edge{}".format(t)]
        out += [p["W1"], p["b1"][None, :], p["W2"], p["b2"][None, :]]
    return out


def _edge_mlp_body(nf_ref, *refs):
    ep, out_ref = refs[:-1], refs[-1]
    x = nf_ref[...]
    for t in range(NT):
        _edge_block(x, ep, t, out_ref)


def _edge_mlp(nf, lp, interpret=False):
    return pl.pallas_call(
        _edge_mlp_body,
        grid=(GRID,),
        in_specs=[pl.BlockSpec((BN, D), lambda j: (j, 0))] + _EDGE_PARAM_SPECS,
        out_specs=pl.BlockSpec((NT, BN, D), lambda j: (0, j, 0)),
        out_shape=jax.ShapeDtypeStruct((NT, N, D), jnp.float32),
        interpret=interpret,
    )(nf, *_edge_args(lp))


def _node_block(nf_ref, a00_ref, a01_ref, a10_ref, a11_ref,
                w1_ref, b1_ref, w2_ref, b2_ref):
    x = jnp.maximum(nf_ref[...], 0.0)
    h = jnp.dot(x, w1_ref[0:D], preferred_element_type=jnp.float32)
    h += jnp.dot(a00_ref[0], w1_ref[D:D + HALF],
                 preferred_element_type=jnp.float32)
    h += jnp.dot(a01_ref[0], w1_ref[D + HALF:2 * D],
                 preferred_element_type=jnp.float32)
    h += jnp.dot(a10_ref[0], w1_ref[2 * D:2 * D + HALF],
                 preferred_element_type=jnp.float32)
    h += jnp.dot(a11_ref[0], w1_ref[2 * D + HALF:3 * D],
                 preferred_element_type=jnp.float32)
    h = jnp.maximum(h + b1_ref[...], 0.0)
    return (jnp.dot(h, w2_ref[...], preferred_element_type=jnp.float32)
            + b2_ref[...])


_AGG_SPECS = [
    pl.BlockSpec((1, BN, HALF), lambda j: (0, j, 0)),
    pl.BlockSpec((1, BN, HALF), lambda j: (1, j, 0)),
    pl.BlockSpec((1, BN, HALF), lambda j: (0, GRID + j, 0)),
    pl.BlockSpec((1, BN, HALF), lambda j: (1, GRID + j, 0)),
]
_NODE_PARAM_SPECS = [
    pl.BlockSpec((3 * D, HH), lambda j: (0, 0)),
    pl.BlockSpec((1, HH), lambda j: (0, 0)),
    pl.BlockSpec((HH, D), lambda j: (0, 0)),
    pl.BlockSpec((1, D), lambda j: (0, 0)),
]


def _node_args(npar):
    return [npar["W1"], npar["b1"][None, :], npar["W2"], npar["b2"][None, :]]


def _node_mlp_body(nf_ref, a00, a01, a10, a11, w1, b1, w2, b2, out_ref):
    out_ref[...] = _node_block(nf_ref, a00, a01, a10, a11, w1, b1, w2, b2)


def _node_mlp(nf, agg, npar, interpret=False):
    return pl.pallas_call(
        _node_mlp_body,
        grid=(GRID,),
        in_specs=([pl.BlockSpec((BN, D), lambda j: (j, 0))] + _AGG_SPECS
                  + _NODE_PARAM_SPECS),
        out_specs=pl.BlockSpec((BN, D), lambda j: (j, 0)),
        out_shape=jax.ShapeDtypeStruct((N, D), jnp.float32),
        interpret=interpret,
    )(nf, agg, agg, agg, agg, *_node_args(npar))


def _fused_body(nf_ref, a00, a01, a10, a11, w1, b1, w2, b2, *refs):
    ep, nf_out, m_out = refs[:-2], refs[-2], refs[-1]
    nf_next = _node_block(nf_ref, a00, a01, a10, a11, w1, b1, w2, b2)
    nf_out[...] = nf_next
    for t in range(NT):
        _edge_block(nf_next, ep, t, m_out)


def _fused_node_edge(nf, agg, npar, lp_next, interpret=False):
    return pl.pallas_call(
        _fused_body,
        grid=(GRID,),
        in_specs=([pl.BlockSpec((BN, D), lambda j: (j, 0))] + _AGG_SPECS
                  + _NODE_PARAM_SPECS + _EDGE_PARAM_SPECS),
        out_specs=[
            pl.BlockSpec((BN, D), lambda j: (j, 0)),
            pl.BlockSpec((NT, BN, D), lambda j: (0, j, 0)),
        ],
        out_shape=[
            jax.ShapeDtypeStruct((N, D), jnp.float32),
            jax.ShapeDtypeStruct((NT, N, D), jnp.float32),
        ],
        interpret=interpret,
    )(nf, agg, agg, agg, agg, *_node_args(npar), *_edge_args(lp_next))


def _sc_agg_body(m4_hbm, gidx_hbm, sidx_hbm, zeros_hbm, out_hbm,
                 agg_sh, gvc, svc, rows, semg):
    c = lax.axis_index("c")
    s = lax.axis_index("s")
    pltpu.sync_copy(zeros_hbm, agg_sh.at[pl.ds(s * ROWS_PER_TILE, ROWS_PER_TILE)])
    plsc.subcore_barrier()

    pltpu.sync_copy(gidx_hbm.at[c, s], gvc)
    pltpu.sync_copy(sidx_hbm.at[s], svc)

    def gather(b, j):
        pltpu.async_copy(m4_hbm.at[gvc.at[b]], rows[j], semg[j])

    def wait_gather(b, j):
        pltpu.make_async_copy(m4_hbm.at[gvc.at[b]], rows[j], semg[j]).wait()

    gather(0, 0)

    def step(b, j):
        j1 = (j + 1) % 2

        @pl.when(b + 1 < NB)
        def _():
            gather(b + 1, j1)

        wait_gather(b, j)
        pltpu.sync_copy(rows[j], agg_sh.at[svc.at[b]], add=True)

    def body(i, carry):
        b0 = 2 * i
        step(b0, 0)
        step(b0 + 1, 1)
        return carry

    lax.fori_loop(0, NB // 2, body, 0)
    plsc.subcore_barrier()
    pltpu.sync_copy(agg_sh.at[pl.ds(s * ROWS_PER_TILE, ROWS_PER_TILE)],
                    out_hbm.at[c, pl.ds(s * ROWS_PER_TILE, ROWS_PER_TILE)])


def _sc_agg(m4, gidx, sidx, zeros):
    mesh = plsc.VectorSubcoreMesh(core_axis_name="c", subcore_axis_name="s")
    kern = pl.kernel(
        _sc_agg_body,
        out_type=jax.ShapeDtypeStruct((NT, SEG, HALF), jnp.float32),
        mesh=mesh,
        scratch_types=[
            pltpu.VMEM_SHARED((SEG, HALF), jnp.float32),
            pltpu.VMEM((NB, K), jnp.int32),
            pltpu.VMEM((NB, K), jnp.int32),
            [pltpu.VMEM((K, HALF), jnp.float32) for _ in range(2)],
            [pltpu.SemaphoreType.DMA for _ in range(2)],
        ],
        compiler_params=pltpu.CompilerParams(use_tc_tiling_on_sc=False),
    )
    return kern(m4, gidx, sidx, zeros)


def kernel(x, edge_index, edge_type, params):
    src = edge_index[0].astype(jnp.int32)
    dst = edge_index[1].astype(jnp.int32)
    et = edge_type.astype(jnp.int32)
    base = et * N + src
    pad = EPT_PAD - PER_TILE
    gidx = jnp.pad(
        jnp.stack([base * 2, base * 2 + 1]).reshape(NT, TILES, PER_TILE),
        ((0, 0), (0, 0), (0, pad))).reshape(NT, TILES, NB, K)
    sidx = jnp.pad((et * N + dst).reshape(TILES, PER_TILE),
                   ((0, 0), (0, pad)),
                   constant_values=NT * N).reshape(TILES, NB, K)
    zeros = jnp.zeros((ROWS_PER_TILE, HALF), jnp.float32)

    nf = x
    n_layers = len(params)
    m = _edge_mlp(nf, params[0])
    for l in range(n_layers):
        m4 = m.reshape(2 * NT * N, HALF)
        agg = _sc_agg(m4, gidx, sidx, zeros)
        if l + 1 < n_layers:
            nf, m = _fused_node_edge(nf, agg, params[l]["node"], params[l + 1])
        else:
            nf = _node_mlp(nf, agg, params[l]["node"])
    return nf
